# baseline (speedup 1.0000x reference)
import jax
import jax.numpy as jnp
from jax.experimental import pallas as pl
from jax.experimental.pallas import tpu as pltpu

B, S, H = 1, 2048, 768
E, K, I = 16, 2, 256
T = B * S


def _silu(v):
    return v * jax.nn.sigmoid(v)


def _moe_kernel(x_ref, wgate_ref, wg_ref, wu_ref, wd_ref, sg_ref, su_ref, sd_ref,
                o_ref, comb_ref):
    e = pl.program_id(0)
    x = x_ref[...]

    @pl.when(e == 0)
    def _():
        logits = jnp.dot(x, wgate_ref[...], preferred_element_type=jnp.float32)
        iota_e = jax.lax.broadcasted_iota(jnp.int32, (T, E), 1)
        a1 = jnp.argmax(logits, axis=-1)
        hot1 = iota_e == a1[:, None]
        m1 = jnp.max(logits, axis=-1, keepdims=True)
        masked = jnp.where(hot1, -jnp.inf, logits)
        a2 = jnp.argmax(masked, axis=-1)
        hot2 = iota_e == a2[:, None]
        m2 = jnp.max(masked, axis=-1, keepdims=True)
        e2 = jnp.exp(m2 - m1)
        w1 = 1.0 / (1.0 + e2)
        w2 = e2 / (1.0 + e2)
        comb_ref[...] = jnp.where(hot1, w1, 0.0) + jnp.where(hot2, w2, 0.0)

    hot_e = (jax.lax.broadcasted_iota(jnp.int32, (1, E), 1) == e).astype(jnp.float32)
    ccol = jnp.sum(comb_ref[...] * hot_e, axis=-1, keepdims=True)

    g = jnp.dot(x, wg_ref[0], preferred_element_type=jnp.float32)
    u = jnp.dot(x, wu_ref[0], preferred_element_type=jnp.float32)
    h = _silu(g) * u * ccol
    y = jnp.dot(h, wd_ref[0], preferred_element_type=jnp.float32)

    @pl.when(e == 0)
    def _():
        gs = jnp.dot(x, sg_ref[...], preferred_element_type=jnp.float32)
        us = jnp.dot(x, su_ref[...], preferred_element_type=jnp.float32)
        hs = _silu(gs) * us
        o_ref[...] = y + jnp.dot(hs, sd_ref[...], preferred_element_type=jnp.float32)

    @pl.when(e != 0)
    def _():
        o_ref[...] += y


def kernel(x, w_gate, wg, wu, wd, sg, su, sd):
    xf = x.reshape(T, H)

    y = pl.pallas_call(
        _moe_kernel,
        grid=(E,),
        in_specs=[
            pl.BlockSpec((T, H), lambda e: (0, 0)),
            pl.BlockSpec((H, E), lambda e: (0, 0)),
            pl.BlockSpec((1, H, I), lambda e: (e, 0, 0)),
            pl.BlockSpec((1, H, I), lambda e: (e, 0, 0)),
            pl.BlockSpec((1, I, H), lambda e: (e, 0, 0)),
            pl.BlockSpec((H, I), lambda e: (0, 0)),
            pl.BlockSpec((H, I), lambda e: (0, 0)),
            pl.BlockSpec((I, H), lambda e: (0, 0)),
        ],
        out_specs=pl.BlockSpec((T, H), lambda e: (0, 0)),
        out_shape=jax.ShapeDtypeStruct((T, H), jnp.float32),
        scratch_shapes=[pltpu.VMEM((T, E), jnp.float32)],
        compiler_params=pltpu.CompilerParams(
            dimension_semantics=("arbitrary",),
        ),
    )(xf, w_gate, wg, wu, wd, sg, su, sd)

    return y.reshape(B, S, H)


# speedup vs baseline: 1.0475x; 1.0475x over previous
import jax
import jax.numpy as jnp
from jax.experimental import pallas as pl
from jax.experimental.pallas import tpu as pltpu

B, S, H = 1, 2048, 768
E, K, I = 16, 2, 256
T = B * S


def _router_kernel(x_ref, wg_ref, comb_ref):
    logits = jnp.dot(x_ref[...], wg_ref[...], preferred_element_type=jnp.float32)
    iota_e = jax.lax.broadcasted_iota(jnp.int32, logits.shape, 1)
    a1 = jnp.argmax(logits, axis=-1)
    hot1 = iota_e == a1[:, None]
    m1 = jnp.max(logits, axis=-1, keepdims=True)
    masked = jnp.where(hot1, -jnp.inf, logits)
    a2 = jnp.argmax(masked, axis=-1)
    hot2 = iota_e == a2[:, None]
    m2 = jnp.max(masked, axis=-1, keepdims=True)
    e2 = jnp.exp(m2 - m1)
    w1 = 1.0 / (1.0 + e2)
    w2 = e2 / (1.0 + e2)
    comb = jnp.where(hot1, w1, 0.0) + jnp.where(hot2, w2, 0.0)
    comb_ref[...] = comb.T.reshape(E, 1, T)


def _silu(v):
    return v * jax.nn.sigmoid(v)


def _moe_kernel(comb_ref, x_ref, wg_ref, wu_ref, wd_ref, sg_ref, su_ref, sd_ref,
                o_ref):
    e = pl.program_id(0)
    x = x_ref[...]
    g = jnp.dot(x, wg_ref[0], preferred_element_type=jnp.float32)
    u = jnp.dot(x, wu_ref[0], preferred_element_type=jnp.float32)
    h = _silu(g) * u * comb_ref[0, 0].reshape(T, 1)
    y = jnp.dot(h, wd_ref[0], preferred_element_type=jnp.float32)

    @pl.when(e == 0)
    def _():
        gs = jnp.dot(x, sg_ref[...], preferred_element_type=jnp.float32)
        us = jnp.dot(x, su_ref[...], preferred_element_type=jnp.float32)
        hs = _silu(gs) * us
        o_ref[...] = y + jnp.dot(hs, sd_ref[...], preferred_element_type=jnp.float32)

    @pl.when(e != 0)
    def _():
        o_ref[...] += y


def kernel(x, w_gate, wg, wu, wd, sg, su, sd):
    xf = x.reshape(T, H)

    comb = pl.pallas_call(
        _router_kernel,
        out_shape=jax.ShapeDtypeStruct((E, 1, T), jnp.float32),
    )(xf, w_gate)

    y = pl.pallas_call(
        _moe_kernel,
        grid=(E,),
        in_specs=[
            pl.BlockSpec((1, 1, T), lambda e: (e, 0, 0)),
            pl.BlockSpec((T, H), lambda e: (0, 0)),
            pl.BlockSpec((1, H, I), lambda e: (e, 0, 0)),
            pl.BlockSpec((1, H, I), lambda e: (e, 0, 0)),
            pl.BlockSpec((1, I, H), lambda e: (e, 0, 0)),
            pl.BlockSpec((H, I), lambda e: (0, 0)),
            pl.BlockSpec((H, I), lambda e: (0, 0)),
            pl.BlockSpec((I, H), lambda e: (0, 0)),
        ],
        out_specs=pl.BlockSpec((T, H), lambda e: (0, 0)),
        out_shape=jax.ShapeDtypeStruct((T, H), jnp.float32),
        compiler_params=pltpu.CompilerParams(
            dimension_semantics=("arbitrary",),
        ),
    )(comb, xf, wg, wu, wd, sg, su, sd)

    return y.reshape(B, S, H)
